# baseline (speedup 1.0000x reference)
import functools

import jax
import jax.numpy as jnp
import numpy as np
from jax import lax
from jax.experimental import pallas as pl
from jax.experimental.pallas import tpu as pltpu
from jax.experimental.pallas import tpu_sc as plsc

BEAM_N = 16
EOS_IDX = 2
EOS_THR = 1.5
NEG_BIG = np.float32(-1e20)
VERY_LOW = np.float32(-3.0e38)

VOCAB = 100000
N_ROWS = 256
N_BATCH = 16
CHUNK = 20000
CHUNKS_PER_ROW = VOCAB // CHUNK
ROWS_PER_TILE = 8
N_CHUNKS = CHUNKS_PER_ROW * ROWS_PER_TILE
BLOCK_V = 50
BLOCKS = CHUNK // (BLOCK_V * 16)


def _merge16(mv, mi, cv, ci):
    sd, sdi = plsc.sort_key_val(cv, ci, descending=True)
    keep = mv >= sd
    nv = jnp.where(keep, mv, sd)
    ni = jnp.where(keep, mi, sdi)
    return plsc.sort_key_val(nv, ni)


def _build(interpret=False):
    mesh = plsc.VectorSubcoreMesh(
        core_axis_name="c", subcore_axis_name="s",
        num_cores=2, num_subcores=16)

    @functools.partial(
        pl.kernel,
        out_type=(
            jax.ShapeDtypeStruct((N_ROWS,), jnp.float32),
            jax.ShapeDtypeStruct((N_ROWS,), jnp.int32),
            jax.ShapeDtypeStruct((N_ROWS,), jnp.int32),
        ),
        mesh=mesh,
        scratch_types=[
            pltpu.VMEM((CHUNK,), jnp.float32),
            pltpu.VMEM((CHUNK,), jnp.float32),
            pltpu.VMEM((N_ROWS,), jnp.float32),
            pltpu.VMEM((16,), jnp.float32),
            pltpu.VMEM((16,), jnp.int32),
            pltpu.VMEM((2 * ROWS_PER_TILE * 16,), jnp.float32),
            pltpu.VMEM((2 * ROWS_PER_TILE * 16,), jnp.int32),
            pltpu.VMEM_SHARED((16 * ROWS_PER_TILE * 16,), jnp.float32),
            pltpu.VMEM_SHARED((16 * ROWS_PER_TILE * 16,), jnp.int32),
            pltpu.SemaphoreType.DMA,
            pltpu.SemaphoreType.DMA,
        ],
        compiler_params=pltpu.CompilerParams(needs_layout_passes=False),
        interpret=interpret,
    )
    def beam_step(lp_hbm, seq_hbm, out_s, out_p, out_t,
                  buf0, buf1, seqv, tmpv, tmpi, locv, loci, shv, shi,
                  sem0, sem1):
        cid = lax.axis_index("c")
        sid = lax.axis_index("s")
        iota = lax.iota(jnp.int32, 16)

        pltpu.sync_copy(seq_hbm, seqv)

        batch = 8 * cid + sid // 2
        row_base = batch * BEAM_N + (sid % 2) * ROWS_PER_TILE
        tile_off = row_base * VOCAB

        def scan_block(buf, boff, ibase, w, use_w):
            macc = jnp.full((16,), VERY_LOW, jnp.float32)
            iacc = jnp.zeros((16,), jnp.int32)
            for j in range(BLOCK_V):
                v = buf[pl.ds(boff + 16 * j, 16)]
                if use_w:
                    v = jnp.where(v < w, v, VERY_LOW)
                idxv = (ibase + 16 * j) + iota
                upd = v > macc
                iacc = jnp.where(upd, idxv, iacc)
                macc = jnp.maximum(macc, v)
            return macc, iacc

        def proc_chunk(buf, c, st):
            ibase0 = (c % CHUNKS_PER_ROW) * CHUNK

            def blk(b, st):
                mv, mi, th, rmx = st
                boff = b * (BLOCK_V * 16)
                ibase = ibase0 + boff
                macc = jnp.full((16,), VERY_LOW, jnp.float32)
                for j in range(BLOCK_V):
                    macc = jnp.maximum(macc, buf[pl.ds(boff + 16 * j, 16)])
                rmx = jnp.maximum(rmx, macc)

                def hit_fn(op):
                    mv, mi, th = op

                    def wcond(cst):
                        return cst[0]

                    def wbody(cst):
                        _, w, mv, mi, th = cst
                        m2, i2 = scan_block(buf, boff, ibase, w, True)
                        again = jnp.any(m2 > th)

                        def m(op2):
                            mv2, mi2 = _merge16(op2[0], op2[1], m2, i2)
                            return mv2, mi2, jnp.min(mv2)

                        mv, mi, th = lax.cond(
                            again, m, lambda o: o, (mv, mi, th))
                        return again, m2, mv, mi, th

                    _, _, mv, mi, th = lax.while_loop(
                        wcond, wbody,
                        (jnp.bool_(True),
                         jnp.full((16,), np.float32(3.0e38), jnp.float32),
                         mv, mi, th))
                    return mv, mi, th

                mv, mi, th = lax.cond(
                    jnp.any(macc > th), hit_fn, lambda o: o, (mv, mi, th))
                return mv, mi, th, rmx

            return lax.fori_loop(0, BLOCKS, blk, st)

        def finalize(row, st):
            mv, mi, th, rmx, eosvec = st
            row_max = jnp.max(rmx)
            eos_val = jnp.max(eosvec)
            masked_eos = jnp.where(
                eos_val > EOS_THR * row_max, eos_val, NEG_BIG)
            cv = jnp.where(iota == 0, masked_eos, VERY_LOW)
            ci = jnp.full((16,), EOS_IDX, jnp.int32)
            mv, mi = _merge16(mv, mi, cv, ci)

            g16 = seqv[pl.ds((row // 16) * 16, 16)]
            sval = jnp.max(jnp.where(iota == row % 16, g16, VERY_LOW))
            tmpv[...] = mv + sval
            tmpi[...] = (row % BEAM_N) * VOCAB + mi
            rr = row - row_base
            pltpu.sync_copy(tmpv, shv.at[pl.ds(sid * 128 + rr * 16, 16)])
            pltpu.sync_copy(tmpi, shi.at[pl.ds(sid * 128 + rr * 16, 16)])

        def new_row(buf, c, st):
            @pl.when(c > 0)
            def _():
                finalize(row_base + c // CHUNKS_PER_ROW - 1, st)
            v0 = buf[pl.ds(0, 16)]
            eosvec = jnp.where(iota == EOS_IDX, v0, VERY_LOW)
            buf[pl.ds(0, 16)] = jnp.where(iota == EOS_IDX, NEG_BIG, v0)
            return (jnp.full((16,), VERY_LOW, jnp.float32),
                    jnp.zeros((16,), jnp.int32),
                    VERY_LOW,
                    v0,
                    eosvec)

        def proc(buf, c, st):
            st = lax.cond(
                c % CHUNKS_PER_ROW == 0,
                lambda a: new_row(buf, c, a),
                lambda a: a,
                st)
            mv, mi, th, rmx = proc_chunk(
                buf, c, (st[0], st[1], st[2], st[3]))
            return (mv, mi, th, rmx, st[4])

        pltpu.async_copy(
            lp_hbm.at[pl.ds(tile_off, CHUNK)], buf0, sem0).wait()

        st0 = (jnp.full((16,), VERY_LOW, jnp.float32),
               jnp.zeros((16,), jnp.int32),
               VERY_LOW,
               jnp.full((16,), VERY_LOW, jnp.float32),
               jnp.full((16,), VERY_LOW, jnp.float32))

        def pair(k, st):
            cp1 = pltpu.async_copy(
                lp_hbm.at[pl.ds(tile_off + (2 * k + 1) * CHUNK, CHUNK)],
                buf1, sem1)
            st = proc(buf0, 2 * k, st)

            @pl.when(k < N_CHUNKS // 2 - 1)
            def _():
                pltpu.async_copy(
                    lp_hbm.at[pl.ds(tile_off + (2 * k + 2) * CHUNK, CHUNK)],
                    buf0, sem0)

            cp1.wait()
            st = proc(buf1, 2 * k + 1, st)

            @pl.when(k < N_CHUNKS // 2 - 1)
            def _():
                pltpu.make_async_copy(
                    lp_hbm.at[pl.ds(0, CHUNK)], buf0, sem0).wait()

            return st

        st = lax.fori_loop(0, N_CHUNKS // 2, pair, st0)
        finalize(row_base + ROWS_PER_TILE - 1, st)

        plsc.subcore_barrier()

        @pl.when(sid < 8)
        def _():
            pltpu.sync_copy(shv.at[pl.ds(2 * sid * 128, 256)], locv)
            pltpu.sync_copy(shi.at[pl.ds(2 * sid * 128, 256)], loci)
            mv = jnp.full((16,), VERY_LOW, jnp.float32)
            mi = jnp.zeros((16,), jnp.int32)
            for r in range(2 * ROWS_PER_TILE):
                mv, mi = _merge16(mv, mi, locv[pl.ds(r * 16, 16)],
                                  loci[pl.ds(r * 16, 16)])
            dv = lax.rev(mv, (0,))
            di = lax.rev(mi, (0,))
            b = 8 * cid + sid
            tmpv[...] = dv
            pltpu.sync_copy(tmpv, out_s.at[pl.ds(b * 16, 16)])
            tmpi[...] = di // VOCAB
            pltpu.sync_copy(tmpi, out_p.at[pl.ds(b * 16, 16)])
            tmpi[...] = di % VOCAB
            pltpu.sync_copy(tmpi, out_t.at[pl.ds(b * 16, 16)])

    return beam_step


_beam_step = _build()


def kernel(log_probs, sequence_scores, step):
    scores, preds, toks = _beam_step(
        jnp.reshape(log_probs, (-1,)), sequence_scores)
    step_f = jnp.asarray(step, jnp.float32)
    return (jnp.reshape(scores, (N_BATCH, BEAM_N)) / step_f,
            jnp.reshape(preds, (N_BATCH, BEAM_N)),
            jnp.reshape(toks, (N_BATCH, BEAM_N)))


# speedup vs baseline: 1.2358x; 1.2358x over previous
import functools

import jax
import jax.numpy as jnp
import numpy as np
from jax import lax
from jax.experimental import pallas as pl
from jax.experimental.pallas import tpu as pltpu
from jax.experimental.pallas import tpu_sc as plsc

BEAM_N = 16
EOS_IDX = 2
EOS_THR = 1.5
NEG_BIG = np.float32(-1e20)
VERY_LOW = np.float32(-3.0e38)

VOCAB = 100000
N_ROWS = 256
N_BATCH = 16
CHUNK = 20000
CHUNKS_PER_ROW = VOCAB // CHUNK
ROWS_PER_TILE = 8
N_CHUNKS = CHUNKS_PER_ROW * ROWS_PER_TILE
BLOCK_V = 50
BLOCKS = CHUNK // (BLOCK_V * 16)


def _merge16(mv, mi, cv, ci):
    sd, sdi = plsc.sort_key_val(cv, ci, descending=True)
    keep = mv >= sd
    nv = jnp.where(keep, mv, sd)
    ni = jnp.where(keep, mi, sdi)
    return plsc.sort_key_val(nv, ni)


def _build(interpret=False):
    mesh = plsc.VectorSubcoreMesh(
        core_axis_name="c", subcore_axis_name="s",
        num_cores=2, num_subcores=16)

    @functools.partial(
        pl.kernel,
        out_type=(
            jax.ShapeDtypeStruct((N_ROWS,), jnp.float32),
            jax.ShapeDtypeStruct((N_ROWS,), jnp.int32),
            jax.ShapeDtypeStruct((N_ROWS,), jnp.int32),
        ),
        mesh=mesh,
        scratch_types=[
            pltpu.VMEM((CHUNK,), jnp.float32),
            pltpu.VMEM((CHUNK,), jnp.float32),
            pltpu.VMEM((N_ROWS,), jnp.float32),
            pltpu.VMEM((16,), jnp.float32),
            pltpu.VMEM((16,), jnp.int32),
            pltpu.VMEM((2 * ROWS_PER_TILE * 16,), jnp.float32),
            pltpu.VMEM((2 * ROWS_PER_TILE * 16,), jnp.int32),
            pltpu.VMEM_SHARED((16 * ROWS_PER_TILE * 16,), jnp.float32),
            pltpu.VMEM_SHARED((16 * ROWS_PER_TILE * 16,), jnp.int32),
            pltpu.SemaphoreType.DMA,
            pltpu.SemaphoreType.DMA,
        ],
        compiler_params=pltpu.CompilerParams(needs_layout_passes=False),
        interpret=interpret,
    )
    def beam_step(lp_hbm, seq_hbm, out_s, out_p, out_t,
                  buf0, buf1, seqv, tmpv, tmpi, locv, loci, shv, shi,
                  sem0, sem1):
        cid = lax.axis_index("c")
        sid = lax.axis_index("s")
        iota = lax.iota(jnp.int32, 16)

        pltpu.sync_copy(seq_hbm, seqv)

        batch = 8 * cid + sid // 2
        row_base = batch * BEAM_N + (sid % 2) * ROWS_PER_TILE
        tile_off = row_base * VOCAB

        def scan_block(buf, boff, ibase, w, use_w):
            macc = jnp.full((16,), VERY_LOW, jnp.float32)
            iacc = jnp.zeros((16,), jnp.int32)
            for j in range(BLOCK_V):
                v = buf[pl.ds(boff + 16 * j, 16)]
                if use_w:
                    v = jnp.where(v < w, v, VERY_LOW)
                idxv = (ibase + 16 * j) + iota
                upd = v > macc
                iacc = jnp.where(upd, idxv, iacc)
                macc = jnp.maximum(macc, v)
            return macc, iacc

        def proc_chunk(buf, c, st):
            ibase0 = (c % CHUNKS_PER_ROW) * CHUNK

            def blk(b, st):
                mv, mi, th, rmx = st
                boff = b * (BLOCK_V * 16)
                ibase = ibase0 + boff
                macc = jnp.full((16,), VERY_LOW, jnp.float32)
                oacc = jnp.zeros((16,), jnp.int32)
                for j in range(BLOCK_V):
                    v = buf[pl.ds(boff + 16 * j, 16)]
                    upd = v > macc
                    oacc = jnp.where(upd, np.int32(j), oacc)
                    macc = jnp.maximum(macc, v)
                rmx = jnp.maximum(rmx, macc)

                def hit_fn(op):
                    mv, mi, th = op
                    iacc = (ibase + oacc * 16) + iota
                    mv, mi = _merge16(mv, mi, macc, iacc)
                    th = jnp.min(mv)

                    def wcond(cst):
                        return cst[0]

                    def wbody(cst):
                        _, w, mv, mi, th = cst
                        m2, i2 = scan_block(buf, boff, ibase, w, True)
                        again = jnp.any(m2 > th)

                        def m(op2):
                            mv2, mi2 = _merge16(op2[0], op2[1], m2, i2)
                            return mv2, mi2, jnp.min(mv2)

                        mv, mi, th = lax.cond(
                            again, m, lambda o: o, (mv, mi, th))
                        return again, m2, mv, mi, th

                    _, _, mv, mi, th = lax.while_loop(
                        wcond, wbody, (jnp.bool_(True), macc, mv, mi, th))
                    return mv, mi, th

                mv, mi, th = lax.cond(
                    jnp.any(macc > th), hit_fn, lambda o: o, (mv, mi, th))
                return mv, mi, th, rmx

            return lax.fori_loop(0, BLOCKS, blk, st)

        def finalize(row, st):
            mv, mi, th, rmx, eosvec = st
            row_max = jnp.max(rmx)
            eos_val = jnp.max(eosvec)
            masked_eos = jnp.where(
                eos_val > EOS_THR * row_max, eos_val, NEG_BIG)
            cv = jnp.where(iota == 0, masked_eos, VERY_LOW)
            ci = jnp.full((16,), EOS_IDX, jnp.int32)
            mv, mi = _merge16(mv, mi, cv, ci)

            g16 = seqv[pl.ds((row // 16) * 16, 16)]
            sval = jnp.max(jnp.where(iota == row % 16, g16, VERY_LOW))
            tmpv[...] = mv + sval
            tmpi[...] = (row % BEAM_N) * VOCAB + mi
            rr = row - row_base
            pltpu.sync_copy(tmpv, shv.at[pl.ds(sid * 128 + rr * 16, 16)])
            pltpu.sync_copy(tmpi, shi.at[pl.ds(sid * 128 + rr * 16, 16)])

        def new_row(buf, c, st):
            @pl.when(c > 0)
            def _():
                finalize(row_base + c // CHUNKS_PER_ROW - 1, st)
            v0 = buf[pl.ds(0, 16)]
            eosvec = jnp.where(iota == EOS_IDX, v0, VERY_LOW)
            buf[pl.ds(0, 16)] = jnp.where(iota == EOS_IDX, NEG_BIG, v0)
            return (jnp.full((16,), VERY_LOW, jnp.float32),
                    jnp.zeros((16,), jnp.int32),
                    VERY_LOW,
                    v0,
                    eosvec)

        def proc(buf, c, st):
            st = lax.cond(
                c % CHUNKS_PER_ROW == 0,
                lambda a: new_row(buf, c, a),
                lambda a: a,
                st)
            mv, mi, th, rmx = proc_chunk(
                buf, c, (st[0], st[1], st[2], st[3]))
            return (mv, mi, th, rmx, st[4])

        pltpu.async_copy(
            lp_hbm.at[pl.ds(tile_off, CHUNK)], buf0, sem0).wait()

        st0 = (jnp.full((16,), VERY_LOW, jnp.float32),
               jnp.zeros((16,), jnp.int32),
               VERY_LOW,
               jnp.full((16,), VERY_LOW, jnp.float32),
               jnp.full((16,), VERY_LOW, jnp.float32))

        def pair(k, st):
            cp1 = pltpu.async_copy(
                lp_hbm.at[pl.ds(tile_off + (2 * k + 1) * CHUNK, CHUNK)],
                buf1, sem1)
            st = proc(buf0, 2 * k, st)

            @pl.when(k < N_CHUNKS // 2 - 1)
            def _():
                pltpu.async_copy(
                    lp_hbm.at[pl.ds(tile_off + (2 * k + 2) * CHUNK, CHUNK)],
                    buf0, sem0)

            cp1.wait()
            st = proc(buf1, 2 * k + 1, st)

            @pl.when(k < N_CHUNKS // 2 - 1)
            def _():
                pltpu.make_async_copy(
                    lp_hbm.at[pl.ds(0, CHUNK)], buf0, sem0).wait()

            return st

        st = lax.fori_loop(0, N_CHUNKS // 2, pair, st0)
        finalize(row_base + ROWS_PER_TILE - 1, st)

        plsc.subcore_barrier()

        @pl.when(sid < 8)
        def _():
            pltpu.sync_copy(shv.at[pl.ds(2 * sid * 128, 256)], locv)
            pltpu.sync_copy(shi.at[pl.ds(2 * sid * 128, 256)], loci)
            mv = jnp.full((16,), VERY_LOW, jnp.float32)
            mi = jnp.zeros((16,), jnp.int32)
            for r in range(2 * ROWS_PER_TILE):
                mv, mi = _merge16(mv, mi, locv[pl.ds(r * 16, 16)],
                                  loci[pl.ds(r * 16, 16)])
            dv = lax.rev(mv, (0,))
            di = lax.rev(mi, (0,))
            b = 8 * cid + sid
            tmpv[...] = dv
            pltpu.sync_copy(tmpv, out_s.at[pl.ds(b * 16, 16)])
            tmpi[...] = di // VOCAB
            pltpu.sync_copy(tmpi, out_p.at[pl.ds(b * 16, 16)])
            tmpi[...] = di % VOCAB
            pltpu.sync_copy(tmpi, out_t.at[pl.ds(b * 16, 16)])

    return beam_step


_beam_step = _build()


def kernel(log_probs, sequence_scores, step):
    scores, preds, toks = _beam_step(
        jnp.reshape(log_probs, (-1,)), sequence_scores)
    step_f = jnp.asarray(step, jnp.float32)
    return (jnp.reshape(scores, (N_BATCH, BEAM_N)) / step_f,
            jnp.reshape(preds, (N_BATCH, BEAM_N)),
            jnp.reshape(toks, (N_BATCH, BEAM_N)))
